# Initial kernel scaffold; baseline (speedup 1.0000x reference)
import jax
import jax.numpy as jnp
from jax.experimental import pallas as pl


def kernel(heatmap_volumes):
    raise NotImplementedError("write your pallas kernel here")


# speedup vs baseline: 6.4544x; 6.4544x over previous
import functools

import jax
import jax.numpy as jnp
from jax.experimental import pallas as pl
from jax.experimental.pallas import tpu as pltpu

X, Y, Z = 128, 128, 64
YZ = Y * Z
K_TOP = 10
NEG_INF = float("-inf")


def _nms_topk_kernel(x_ref, out_ref, v_scr):
    x = x_ref[0]

    lane = jax.lax.broadcasted_iota(jnp.int32, (X, YZ), 1)
    zpos = jax.lax.rem(lane, Z)

    ninf_col = jnp.full((X, 1), NEG_INF, dtype=jnp.float32)
    xp = jnp.concatenate([ninf_col, x[:, :-1]], axis=1)
    xm = jnp.concatenate([x[:, 1:], ninf_col], axis=1)
    mz = jnp.maximum(
        x,
        jnp.maximum(
            jnp.where(zpos == 0, NEG_INF, xp),
            jnp.where(zpos == Z - 1, NEG_INF, xm),
        ),
    )

    ninf_y = jnp.full((X, Z), NEG_INF, dtype=jnp.float32)
    yp = jnp.concatenate([ninf_y, mz[:, :-Z]], axis=1)
    ym = jnp.concatenate([mz[:, Z:], ninf_y], axis=1)
    my = jnp.maximum(mz, jnp.maximum(yp, ym))

    ninf_row = jnp.full((1, YZ), NEG_INF, dtype=jnp.float32)
    rp = jnp.concatenate([ninf_row, my[:-1, :]], axis=0)
    rm = jnp.concatenate([my[1:, :], ninf_row], axis=0)
    mfull = jnp.maximum(my, jnp.maximum(rp, rm))

    v = jnp.where(x == mfull, x, 0.0)
    v_scr[...] = v
    m_rows = jnp.max(v, axis=1, keepdims=True)

    row_iota = jax.lax.broadcasted_iota(jnp.int32, (X, 1), 0)
    col_iota = jax.lax.broadcasted_iota(jnp.int32, (1, YZ), 1)
    out_lane = jax.lax.broadcasted_iota(jnp.int32, (1, 128), 1)

    for k in range(K_TOP):
        m0 = jnp.max(m_rows)
        r_star = jnp.min(jnp.where(m_rows == m0, row_iota, X))
        row = v_scr[pl.ds(r_star, 1), :]
        c_star = jnp.min(jnp.where(row == m0, col_iota, YZ))
        new_row = jnp.where(col_iota == c_star, -1.0, row)
        v_scr[pl.ds(r_star, 1), :] = new_row
        m_rows = jnp.where(row_iota == r_star, jnp.max(new_row), m_rows)

        ix = r_star.astype(jnp.float32)
        iy = (c_star // Z).astype(jnp.float32)
        iz = (c_star % Z).astype(jnp.float32)
        fx = ix / (X - 1.0) * 8000.0 + 0.0 - 4000.0
        fy = iy / (Y - 1.0) * 8000.0 + 0.0 - 4000.0
        fz = iz / (Z - 1.0) * 2000.0 + 800.0 - 1000.0
        rowv = jnp.where(
            out_lane == 0,
            fx,
            jnp.where(
                out_lane == 1,
                fy,
                jnp.where(out_lane == 2, fz, jnp.where(out_lane == 4, m0, 0.0)),
            ),
        )
        out_ref[0, pl.ds(k, 1), :] = rowv


@functools.partial(jax.jit, static_argnums=())
def kernel(heatmap_volumes):
    b = heatmap_volumes.shape[0]
    hv = heatmap_volumes.reshape(b, X, YZ)
    out = pl.pallas_call(
        _nms_topk_kernel,
        grid=(b,),
        in_specs=[pl.BlockSpec((1, X, YZ), lambda i: (i, 0, 0))],
        out_specs=pl.BlockSpec((1, 16, 128), lambda i: (i, 0, 0)),
        out_shape=jax.ShapeDtypeStruct((b, 16, 128), jnp.float32),
        scratch_shapes=[pltpu.VMEM((X, YZ), jnp.float32)],
        compiler_params=pltpu.CompilerParams(
            dimension_semantics=("arbitrary",),
        ),
    )(hv)
    return out[:, :K_TOP, :5]
